# baseline (speedup 1.0000x reference)
import functools
import jax
import jax.numpy as jnp
from jax import lax
from jax.experimental import pallas as pl
from jax.experimental.pallas import tpu as pltpu
from jax.experimental.pallas import tpu_sc as plsc

N = 10000
D = 128
C = 7
E = 320000

NC = 2
NS = 16
NW = NC * NS
CH = 128
K0 = 48
K1 = 112
E_PAD = NS * (K0 + K1) * CH
N_PAD = 10112
R = N_PAD // NS

_f32 = jnp.float32


BK = 8


def _chunk_range(c, s):
    base = jnp.where(c == 0, s * K0, NS * K0 + s * K1)
    nblocks = jnp.where(c == 0, K0 // BK, K1 // BK)
    return base, nblocks


def _sc_body_cnt(dst_hbm, ones_hbm, z128_hbm, out_hbm,
                 accum, dst_v, ones_v):
    c = lax.axis_index("c")
    s = lax.axis_index("s")
    base, nblocks = _chunk_range(c, s)
    pltpu.sync_copy(z128_hbm, accum.at[pl.ds(s * R, R)])
    pltpu.sync_copy(ones_hbm, ones_v)
    plsc.subcore_barrier()

    def blk(b, carry):
        pltpu.sync_copy(dst_hbm.at[pl.ds(base + b * BK, BK)], dst_v)
        for j in range(BK):
            pltpu.sync_copy(ones_v, accum.at[dst_v.at[j]], add=True)
        return carry

    lax.fori_loop(0, nblocks, blk, 0)
    plsc.subcore_barrier()
    pltpu.sync_copy(accum.at[pl.ds(s * R, R)], out_hbm.at[c, pl.ds(s * R, R)])


def _sc_body(y_hbm, src_hbm, dst_hbm, z128_hbm,
             out_hbm, accum, src_v, dst_v, rows_a, rows_b, sem_a, sem_b):
    c = lax.axis_index("c")
    s = lax.axis_index("s")
    base, nblocks = _chunk_range(c, s)
    pltpu.sync_copy(z128_hbm, accum.at[pl.ds(s * R, R)])
    plsc.subcore_barrier()
    rows = (rows_a, rows_b)
    sems = (sem_a, sem_b)

    def blk(b, carry):
        pltpu.sync_copy(src_hbm.at[pl.ds(base + b * BK, BK)], src_v)
        pltpu.sync_copy(dst_hbm.at[pl.ds(base + b * BK, BK)], dst_v)
        pending = [pltpu.async_copy(y_hbm.at[src_v.at[0]], rows[0], sems[0])]
        for j in range(BK):
            if j + 1 < BK:
                pending.append(
                    pltpu.async_copy(y_hbm.at[src_v.at[j + 1]],
                                     rows[(j + 1) % 2], sems[(j + 1) % 2]))
            pending[j].wait()
            pltpu.sync_copy(rows[j % 2], accum.at[dst_v.at[j]], add=True)
        return carry

    lax.fori_loop(0, nblocks, blk, 0)
    plsc.subcore_barrier()
    pltpu.sync_copy(accum.at[pl.ds(s * R, R)], out_hbm.at[c, pl.ds(s * R, R)])


_SC_MESH = plsc.VectorSubcoreMesh(core_axis_name="c", subcore_axis_name="s",
                                  num_cores=NC, num_subcores=NS)

_seg_cnt = pl.kernel(
    _sc_body_cnt,
    out_type=jax.ShapeDtypeStruct((NC, N_PAD, 128), _f32),
    mesh=_SC_MESH,
    scratch_types=[
        pltpu.VMEM_SHARED((N_PAD, 128), _f32),
        pltpu.VMEM((BK, CH), jnp.int32),
        pltpu.VMEM((CH, 128), _f32),
    ],
)

_seg_sum = pl.kernel(
    _sc_body,
    out_type=jax.ShapeDtypeStruct((NC, N_PAD, 128), _f32),
    mesh=_SC_MESH,
    scratch_types=[
        pltpu.VMEM_SHARED((N_PAD, 128), _f32),
        pltpu.VMEM((BK, CH), jnp.int32),
        pltpu.VMEM((BK, CH), jnp.int32),
        pltpu.VMEM((CH, 128), _f32),
        pltpu.VMEM((CH, 128), _f32),
        pltpu.SemaphoreType.DMA,
        pltpu.SemaphoreType.DMA,
    ],
)


def _tc0_body(x_ref, wl_ref, wr_ref, b_ref, y_ref, r_ref):
    x = x_ref[...]
    y_ref[...] = jnp.dot(x, wl_ref[...].T, preferred_element_type=_f32)
    r_ref[...] = jnp.dot(x, wr_ref[...].T, preferred_element_type=_f32) + b_ref[...]


def _bn_relu(h, g, b, eps=1e-5):
    m = jnp.mean(h, axis=0, keepdims=True)
    v = jnp.mean((h - m) * (h - m), axis=0, keepdims=True)
    return jax.nn.relu((h - m) * jax.lax.rsqrt(v + eps) * g + b)


def _tc1_body(sp_ref, degp_ref, r_ref, g_ref, bb_ref, wl_ref, wr_ref, b2_ref,
              y_ref, r2_ref, dinv_ref):
    deg = degp_ref[0, :N, 0:1] + degp_ref[1, :N, 0:1]
    dinv = 1.0 / jnp.maximum(deg, 1.0)
    dinv_ref[...] = dinv
    s = sp_ref[0, :N, :] + sp_ref[1, :N, :]
    h = _bn_relu(s * dinv + r_ref[...], g_ref[...], bb_ref[...])
    y_ref[...] = jnp.dot(h, wl_ref[...].T, preferred_element_type=_f32)
    r2_ref[...] = jnp.dot(h, wr_ref[...].T, preferred_element_type=_f32) + b2_ref[...]


def _tc2_body(sp_ref, dinv_ref, r_ref, g_ref, bb_ref, wl_ref, wr_ref, b3_ref,
              y_ref, r3_ref):
    s = sp_ref[0, :N, :] + sp_ref[1, :N, :]
    h = _bn_relu(s * dinv_ref[...] + r_ref[...], g_ref[...], bb_ref[...])
    y_ref[...] = jnp.dot(h, wl_ref[...].T, preferred_element_type=_f32)
    r3_ref[...] = jnp.dot(h, wr_ref[...].T, preferred_element_type=_f32) + b3_ref[...]


def _tc3_body(sp_ref, dinv_ref, r_ref, d1w_ref, d1b_ref, d2w_ref, d2b_ref,
              d3w_ref, d3b_ref, o1_ref, o2_ref, o3_ref):
    s = sp_ref[0, :N, :] + sp_ref[1, :N, :]
    h = s * dinv_ref[...] + r_ref[...]
    o1_ref[...] = jnp.dot(h, d1w_ref[...].T, preferred_element_type=_f32) + d1b_ref[...]
    o2_ref[...] = jnp.dot(h, d2w_ref[...].T, preferred_element_type=_f32) + d2b_ref[...]
    o3_ref[...] = jnp.dot(h, d3w_ref[...].T, preferred_element_type=_f32) + d3b_ref[...]


def _tc_call(body, out_shapes):
    return pl.pallas_call(
        body,
        out_shape=[jax.ShapeDtypeStruct(s, _f32) for s in out_shapes],
    )


def kernel(x, edge_index, Wl1, Wr1, b1, Wl2, Wr2, b2, Wl3, Wr3, b3,
           bn1_g, bn1_b, bn2_g, bn2_b, d1_W, d1_b, d2_W, d2_b, d3_W, d3_b):
    src = edge_index[0].astype(jnp.int32)
    dst = edge_index[1].astype(jnp.int32)
    pad = E_PAD - E
    nch = E_PAD // CH
    src_p = jnp.concatenate([src, jnp.zeros((pad,), jnp.int32)]).reshape(nch, CH)
    dst_p = jnp.concatenate([dst, jnp.full((pad,), N, jnp.int32)]).reshape(nch, CH)
    z128 = jnp.zeros((R, 128), _f32)
    ones128 = jnp.ones((CH, 128), _f32)

    b1r = b1.reshape(1, 128)
    b2r = b2.reshape(1, 128)
    b3r = b3.reshape(1, 128)
    g1 = bn1_g.reshape(1, 128)
    bb1 = bn1_b.reshape(1, 128)
    g2 = bn2_g.reshape(1, 128)
    bb2 = bn2_b.reshape(1, 128)
    d1b = d1_b.reshape(1, 128)
    d2b = d2_b.reshape(1, C)
    d3b = d3_b.reshape(1, 128)

    degp = _seg_cnt(dst_p, ones128, z128)
    y1, r1 = _tc_call(_tc0_body, [(N, 128), (N, 128)])(x, Wl1, Wr1, b1r)
    s1 = _seg_sum(y1, src_p, dst_p, z128)
    y2, r2, dinv = _tc_call(_tc1_body, [(N, 128), (N, 128), (N, 1)])(
        s1, degp, r1, g1, bb1, Wl2, Wr2, b2r)
    s2 = _seg_sum(y2, src_p, dst_p, z128)
    y3, r3 = _tc_call(_tc2_body, [(N, 128), (N, 128)])(
        s2, dinv, r2, g2, bb2, Wl3, Wr3, b3r)
    s3 = _seg_sum(y3, src_p, dst_p, z128)
    o1, o2, o3 = _tc_call(_tc3_body, [(N, 128), (N, C), (N, 128)])(
        s3, dinv, r3, d1_W, d1b, d2_W, d2b, d3_W, d3b)
    return (o1, o2, o3)


# speedup vs baseline: 1.1178x; 1.1178x over previous
import functools
import jax
import jax.numpy as jnp
from jax import lax
from jax.experimental import pallas as pl
from jax.experimental.pallas import tpu as pltpu
from jax.experimental.pallas import tpu_sc as plsc

N = 10000
D = 128
C = 7
E = 320000

NC = 2
NS = 16
NW = NC * NS
CH = 128
K0 = 112
K1 = 48
E_PAD = NS * (K0 + K1) * CH
N_PAD = 10112
R = N_PAD // NS

_f32 = jnp.float32


BK = 8


def _chunk_range(c, s):
    base = jnp.where(c == 0, s * K0, NS * K0 + s * K1)
    nblocks = jnp.where(c == 0, K0 // BK, K1 // BK)
    return base, nblocks


def _sc_body_cnt(dst_hbm, ones_hbm, z128_hbm, out_hbm,
                 accum, dst_v, ones_v):
    c = lax.axis_index("c")
    s = lax.axis_index("s")
    base, nblocks = _chunk_range(c, s)
    pltpu.sync_copy(z128_hbm, accum.at[pl.ds(s * R, R)])
    pltpu.sync_copy(ones_hbm, ones_v)
    plsc.subcore_barrier()

    def blk(b, carry):
        pltpu.sync_copy(dst_hbm.at[pl.ds(base + b * BK, BK)], dst_v)
        for j in range(BK):
            pltpu.sync_copy(ones_v, accum.at[dst_v.at[j]], add=True)
        return carry

    lax.fori_loop(0, nblocks, blk, 0)
    plsc.subcore_barrier()
    pltpu.sync_copy(accum.at[pl.ds(s * R, R)], out_hbm.at[c, pl.ds(s * R, R)])


def _sc_body(y_hbm, src_hbm, dst_hbm, z128_hbm,
             out_hbm, accum, src_v, dst_v, rows_a, rows_b, sem_a, sem_b):
    c = lax.axis_index("c")
    s = lax.axis_index("s")
    base, nblocks = _chunk_range(c, s)
    pltpu.sync_copy(z128_hbm, accum.at[pl.ds(s * R, R)])
    plsc.subcore_barrier()
    rows = (rows_a, rows_b)
    sems = (sem_a, sem_b)

    def blk(b, carry):
        pltpu.sync_copy(src_hbm.at[pl.ds(base + b * BK, BK)], src_v)
        pltpu.sync_copy(dst_hbm.at[pl.ds(base + b * BK, BK)], dst_v)
        pending = [pltpu.async_copy(y_hbm.at[src_v.at[0]], rows[0], sems[0])]
        for j in range(BK):
            if j + 1 < BK:
                pending.append(
                    pltpu.async_copy(y_hbm.at[src_v.at[j + 1]],
                                     rows[(j + 1) % 2], sems[(j + 1) % 2]))
            pending[j].wait()
            pltpu.sync_copy(rows[j % 2], accum.at[dst_v.at[j]], add=True)
        return carry

    lax.fori_loop(0, nblocks, blk, 0)
    plsc.subcore_barrier()
    pltpu.sync_copy(accum.at[pl.ds(s * R, R)], out_hbm.at[c, pl.ds(s * R, R)])


_SC_MESH = plsc.VectorSubcoreMesh(core_axis_name="c", subcore_axis_name="s",
                                  num_cores=NC, num_subcores=NS)

_seg_cnt = pl.kernel(
    _sc_body_cnt,
    out_type=jax.ShapeDtypeStruct((NC, N_PAD, 128), _f32),
    mesh=_SC_MESH,
    scratch_types=[
        pltpu.VMEM_SHARED((N_PAD, 128), _f32),
        pltpu.VMEM((BK, CH), jnp.int32),
        pltpu.VMEM((CH, 128), _f32),
    ],
)

_seg_sum = pl.kernel(
    _sc_body,
    out_type=jax.ShapeDtypeStruct((NC, N_PAD, 128), _f32),
    mesh=_SC_MESH,
    scratch_types=[
        pltpu.VMEM_SHARED((N_PAD, 128), _f32),
        pltpu.VMEM((BK, CH), jnp.int32),
        pltpu.VMEM((BK, CH), jnp.int32),
        pltpu.VMEM((CH, 128), _f32),
        pltpu.VMEM((CH, 128), _f32),
        pltpu.SemaphoreType.DMA,
        pltpu.SemaphoreType.DMA,
    ],
)


def _tc0_body(x_ref, wl_ref, wr_ref, b_ref, y_ref, r_ref):
    x = x_ref[...]
    y_ref[...] = jnp.dot(x, wl_ref[...].T, preferred_element_type=_f32)
    r_ref[...] = jnp.dot(x, wr_ref[...].T, preferred_element_type=_f32) + b_ref[...]


def _bn_relu(h, g, b, eps=1e-5):
    m = jnp.mean(h, axis=0, keepdims=True)
    v = jnp.mean((h - m) * (h - m), axis=0, keepdims=True)
    return jax.nn.relu((h - m) * jax.lax.rsqrt(v + eps) * g + b)


def _tc1_body(sp_ref, degp_ref, r_ref, g_ref, bb_ref, wl_ref, wr_ref, b2_ref,
              y_ref, r2_ref, dinv_ref):
    deg = degp_ref[0, :N, 0:1] + degp_ref[1, :N, 0:1]
    dinv = 1.0 / jnp.maximum(deg, 1.0)
    dinv_ref[...] = dinv
    s = sp_ref[0, :N, :] + sp_ref[1, :N, :]
    h = _bn_relu(s * dinv + r_ref[...], g_ref[...], bb_ref[...])
    y_ref[...] = jnp.dot(h, wl_ref[...].T, preferred_element_type=_f32)
    r2_ref[...] = jnp.dot(h, wr_ref[...].T, preferred_element_type=_f32) + b2_ref[...]


def _tc2_body(sp_ref, dinv_ref, r_ref, g_ref, bb_ref, wl_ref, wr_ref, b3_ref,
              y_ref, r3_ref):
    s = sp_ref[0, :N, :] + sp_ref[1, :N, :]
    h = _bn_relu(s * dinv_ref[...] + r_ref[...], g_ref[...], bb_ref[...])
    y_ref[...] = jnp.dot(h, wl_ref[...].T, preferred_element_type=_f32)
    r3_ref[...] = jnp.dot(h, wr_ref[...].T, preferred_element_type=_f32) + b3_ref[...]


def _tc3_body(sp_ref, dinv_ref, r_ref, d1w_ref, d1b_ref, d2w_ref, d2b_ref,
              d3w_ref, d3b_ref, o1_ref, o2_ref, o3_ref):
    s = sp_ref[0, :N, :] + sp_ref[1, :N, :]
    h = s * dinv_ref[...] + r_ref[...]
    o1_ref[...] = jnp.dot(h, d1w_ref[...].T, preferred_element_type=_f32) + d1b_ref[...]
    o2_ref[...] = jnp.dot(h, d2w_ref[...].T, preferred_element_type=_f32) + d2b_ref[...]
    o3_ref[...] = jnp.dot(h, d3w_ref[...].T, preferred_element_type=_f32) + d3b_ref[...]


def _tc_call(body, out_shapes):
    return pl.pallas_call(
        body,
        out_shape=[jax.ShapeDtypeStruct(s, _f32) for s in out_shapes],
    )


def kernel(x, edge_index, Wl1, Wr1, b1, Wl2, Wr2, b2, Wl3, Wr3, b3,
           bn1_g, bn1_b, bn2_g, bn2_b, d1_W, d1_b, d2_W, d2_b, d3_W, d3_b):
    src = edge_index[0].astype(jnp.int32)
    dst = edge_index[1].astype(jnp.int32)
    pad = E_PAD - E
    nch = E_PAD // CH
    src_p = jnp.concatenate([src, jnp.zeros((pad,), jnp.int32)]).reshape(nch, CH)
    dst_p = jnp.concatenate([dst, jnp.full((pad,), N, jnp.int32)]).reshape(nch, CH)
    z128 = jnp.zeros((R, 128), _f32)
    ones128 = jnp.ones((CH, 128), _f32)

    b1r = b1.reshape(1, 128)
    b2r = b2.reshape(1, 128)
    b3r = b3.reshape(1, 128)
    g1 = bn1_g.reshape(1, 128)
    bb1 = bn1_b.reshape(1, 128)
    g2 = bn2_g.reshape(1, 128)
    bb2 = bn2_b.reshape(1, 128)
    d1b = d1_b.reshape(1, 128)
    d2b = d2_b.reshape(1, C)
    d3b = d3_b.reshape(1, 128)

    degp = _seg_cnt(dst_p, ones128, z128)
    y1, r1 = _tc_call(_tc0_body, [(N, 128), (N, 128)])(x, Wl1, Wr1, b1r)
    s1 = _seg_sum(y1, src_p, dst_p, z128)
    y2, r2, dinv = _tc_call(_tc1_body, [(N, 128), (N, 128), (N, 1)])(
        s1, degp, r1, g1, bb1, Wl2, Wr2, b2r)
    s2 = _seg_sum(y2, src_p, dst_p, z128)
    y3, r3 = _tc_call(_tc2_body, [(N, 128), (N, 128)])(
        s2, dinv, r2, g2, bb2, Wl3, Wr3, b3r)
    s3 = _seg_sum(y3, src_p, dst_p, z128)
    o1, o2, o3 = _tc_call(_tc3_body, [(N, 128), (N, C), (N, 128)])(
        s3, dinv, r3, d1_W, d1b, d2_W, d2b, d3_W, d3b)
    return (o1, o2, o3)
